# baseline (speedup 1.0000x reference)
import math

import jax
import jax.numpy as jnp
import numpy as np
from jax.experimental import pallas as pl
from jax.experimental.pallas import tpu as pltpu

NUM_NODES = 160
SEQ_LEN = 128
BATCH = 8
TOPK = 20

_DINV = np.float32(np.float32(20.0) ** np.float32(-0.5))
_NORM = np.float32(_DINV * _DINV)
_NEG = np.float32(-3.0e38)

_GUMBEL = np.asarray(
    jax.random.gumbel(jax.random.key(42), (NUM_NODES * NUM_NODES, 2), jnp.float32)
).reshape(BATCH, NUM_NODES, 2 * TOPK)


def _fused_kernel(emb_ref, lg_ref, gu_ref, x_ref, w_ref, b_ref, o_ref, a_scr):
    i = pl.program_id(0)

    @pl.when(i == 0)
    def _build_adjacency():
        emb = emb_ref[...]
        dot = jax.lax.dot_general(
            emb, emb, (((1,), (1,)), ((), ())), preferred_element_type=jnp.float32
        )
        row_i = jax.lax.broadcasted_iota(jnp.int32, (NUM_NODES, NUM_NODES), 0)
        col_i = jax.lax.broadcasted_iota(jnp.int32, (NUM_NODES, NUM_NODES), 1)
        eye = (row_i == col_i).astype(jnp.float32)
        n2_col = jnp.sum(emb * emb, axis=1, keepdims=True)
        n2_row = jnp.max(eye * n2_col, axis=0, keepdims=True)
        cos = dot / (jnp.sqrt(n2_col) * jnp.sqrt(n2_row))

        s = lg_ref[...] + gu_ref[...]

        acc = jnp.zeros((BATCH, NUM_NODES, NUM_NODES), jnp.float32)
        cosm = cos
        for t in range(TOPK):
            mx = jnp.max(cosm, axis=1, keepdims=True)
            jstar = jnp.min(
                jnp.where(cosm >= mx, col_i, np.int32(NUM_NODES)),
                axis=1, keepdims=True,
            )
            m = (col_i == jstar).astype(jnp.float32)
            gate_t = (s[:, :, 2 * t : 2 * t + 1] >= s[:, :, 2 * t + 1 : 2 * t + 2])
            acc = acc + gate_t.astype(jnp.float32) * m[None, :, :]
            cosm = jnp.where(m > 0.0, _NEG, cosm)
        a_scr[...] = acc * _NORM

    xb = x_ref[0]
    w = w_ref[...]
    h = jax.lax.dot_general(
        w, xb, (((0,), (0,)), ((), ())), preferred_element_type=jnp.float32
    )
    bias = b_ref[...]
    for bn in range(BATCH):
        hb = h[:, bn * NUM_NODES : (bn + 1) * NUM_NODES]
        ob = jax.lax.dot_general(
            hb, a_scr[bn], (((1,), (1,)), ((), ())),
            preferred_element_type=jnp.float32,
        )
        o_ref[0, :, bn * NUM_NODES : (bn + 1) * NUM_NODES] = ob + bias


def kernel(x, emb_table, weight, bias, logits):
    n_total = BATCH * NUM_NODES
    lg = logits.reshape(BATCH, NUM_NODES, 2 * TOPK)

    zero3 = lambda i: (0, 0, 0)
    out = pl.pallas_call(
        _fused_kernel,
        grid=(BATCH,),
        in_specs=[
            pl.BlockSpec((NUM_NODES, 64), lambda i: (0, 0)),
            pl.BlockSpec((BATCH, NUM_NODES, 2 * TOPK), zero3),
            pl.BlockSpec((BATCH, NUM_NODES, 2 * TOPK), zero3),
            pl.BlockSpec((1, SEQ_LEN, n_total), lambda i: (i, 0, 0)),
            pl.BlockSpec((SEQ_LEN, SEQ_LEN), lambda i: (0, 0)),
            pl.BlockSpec((SEQ_LEN, 1), lambda i: (0, 0)),
        ],
        out_specs=pl.BlockSpec((1, SEQ_LEN, n_total), lambda i: (i, 0, 0)),
        out_shape=jax.ShapeDtypeStruct((BATCH, SEQ_LEN, n_total), jnp.float32),
        scratch_shapes=[pltpu.VMEM((BATCH, NUM_NODES, NUM_NODES), jnp.float32)],
    )(emb_table, lg, jnp.asarray(_GUMBEL), x, weight, bias.reshape(SEQ_LEN, 1))
    return out


# speedup vs baseline: 1.5063x; 1.5063x over previous
import math

import jax
import jax.numpy as jnp
import numpy as np
from jax.experimental import pallas as pl
from jax.experimental.pallas import tpu as pltpu

NUM_NODES = 160
SEQ_LEN = 128
BATCH = 8
TOPK = 20

_DINV = np.float32(np.float32(20.0) ** np.float32(-0.5))
_NORM = np.float32(_DINV * _DINV)
_NEG = np.float32(-3.0e38)

_G = np.asarray(
    jax.random.gumbel(jax.random.key(42), (NUM_NODES * NUM_NODES, 2), jnp.float32)
)
_GD = (_G[:, 1] - _G[:, 0]).reshape(BATCH, NUM_NODES, TOPK)


def _fused_kernel(emb_ref, dl_ref, gd_ref, x_ref, w_ref, b_ref, o_ref, a_scr):
    i = pl.program_id(0)

    @pl.when(i == 0)
    def _build_adjacency():
        emb = emb_ref[...]
        dot = jax.lax.dot_general(
            emb, emb, (((1,), (1,)), ((), ())), preferred_element_type=jnp.float32
        )
        row_i = jax.lax.broadcasted_iota(jnp.int32, (NUM_NODES, NUM_NODES), 0)
        col_i = jax.lax.broadcasted_iota(jnp.int32, (NUM_NODES, NUM_NODES), 1)
        eye = (row_i == col_i).astype(jnp.float32)
        n2_col = jnp.sum(emb * emb, axis=1, keepdims=True)
        n2_row = jnp.max(eye * n2_col, axis=0, keepdims=True)
        cos = dot / (jnp.sqrt(n2_col) * jnp.sqrt(n2_row))

        gate = (dl_ref[...] >= gd_ref[...]).astype(jnp.float32)

        acc = jnp.zeros((BATCH, NUM_NODES, NUM_NODES), jnp.float32)
        cosm = cos
        for t in range(TOPK):
            mx = jnp.max(cosm, axis=1, keepdims=True)
            jstar = jnp.min(
                jnp.where(cosm >= mx, col_i, np.int32(NUM_NODES)),
                axis=1, keepdims=True,
            )
            m = (col_i == jstar).astype(jnp.float32)
            acc = acc + gate[:, :, t][:, :, None] * m[None, :, :]
            cosm = jnp.where(m > 0.0, _NEG, cosm)
        a_scr[...] = acc * _NORM

    xb = x_ref[0]
    w = w_ref[...]
    h = jax.lax.dot_general(
        w, xb, (((0,), (0,)), ((), ())), preferred_element_type=jnp.float32
    )
    bias = b_ref[...]
    for bn in range(BATCH):
        hb = h[:, bn * NUM_NODES : (bn + 1) * NUM_NODES]
        ob = jax.lax.dot_general(
            hb, a_scr[bn], (((1,), (1,)), ((), ())),
            preferred_element_type=jnp.float32,
        )
        o_ref[0, :, bn * NUM_NODES : (bn + 1) * NUM_NODES] = ob + bias


def kernel(x, emb_table, weight, bias, logits):
    n_total = BATCH * NUM_NODES
    dl = (logits[:, 0] - logits[:, 1]).reshape(BATCH, NUM_NODES, TOPK)

    zero3 = lambda i: (0, 0, 0)
    out = pl.pallas_call(
        _fused_kernel,
        grid=(BATCH,),
        in_specs=[
            pl.BlockSpec((NUM_NODES, 64), lambda i: (0, 0)),
            pl.BlockSpec((BATCH, NUM_NODES, TOPK), zero3),
            pl.BlockSpec((BATCH, NUM_NODES, TOPK), zero3),
            pl.BlockSpec((1, SEQ_LEN, n_total), lambda i: (i, 0, 0)),
            pl.BlockSpec((SEQ_LEN, SEQ_LEN), lambda i: (0, 0)),
            pl.BlockSpec((SEQ_LEN, 1), lambda i: (0, 0)),
        ],
        out_specs=pl.BlockSpec((1, SEQ_LEN, n_total), lambda i: (i, 0, 0)),
        out_shape=jax.ShapeDtypeStruct((BATCH, SEQ_LEN, n_total), jnp.float32),
        scratch_shapes=[pltpu.VMEM((BATCH, NUM_NODES, NUM_NODES), jnp.float32)],
    )(emb_table, dl, jnp.asarray(_GD), x, weight, bias.reshape(SEQ_LEN, 1))
    return out


# speedup vs baseline: 1.6186x; 1.0746x over previous
import math

import jax
import jax.numpy as jnp
import numpy as np
from jax.experimental import pallas as pl
from jax.experimental.pallas import tpu as pltpu

NUM_NODES = 160
SEQ_LEN = 128
BATCH = 8
TOPK = 20

_DINV = np.float32(np.float32(20.0) ** np.float32(-0.5))
_NORM = np.float32(_DINV * _DINV)
_NEG = np.float32(-3.0e38)

_G = np.asarray(
    jax.random.gumbel(jax.random.key(42), (NUM_NODES * NUM_NODES, 2), jnp.float32)
)
_GD = (_G[:, 1] - _G[:, 0]).reshape(BATCH, NUM_NODES, TOPK)


def _fused_kernel(emb_ref, dl_ref, gd_ref, x_ref, w_ref, b_ref, o_ref, a_scr):
    i = pl.program_id(0)

    @pl.when(i == 0)
    def _build_adjacency():
        emb = emb_ref[...]
        dot = jax.lax.dot_general(
            emb, emb, (((1,), (1,)), ((), ())), preferred_element_type=jnp.float32
        )
        row_i = jax.lax.broadcasted_iota(jnp.int32, (NUM_NODES, NUM_NODES), 0)
        col_i = jax.lax.broadcasted_iota(jnp.int32, (NUM_NODES, NUM_NODES), 1)
        eye = (row_i == col_i).astype(jnp.float32)
        n2_col = jnp.sum(emb * emb, axis=1, keepdims=True)
        n2_row = jnp.max(eye * n2_col, axis=0, keepdims=True)
        cos = dot / (jnp.sqrt(n2_col) * jnp.sqrt(n2_row))

        gate = (dl_ref[...] >= gd_ref[...]).astype(jnp.int32)
        gbits = jnp.zeros((NUM_NODES, TOPK), jnp.int32)
        for b in range(BATCH):
            gbits = gbits + gate[b] * np.int32(1 << b)

        acc = jnp.zeros((NUM_NODES, NUM_NODES), jnp.int32)
        cosm = cos
        for t in range(TOPK):
            mx = jnp.max(cosm, axis=1, keepdims=True)
            jstar = jnp.min(
                jnp.where(cosm >= mx, col_i, np.int32(NUM_NODES)),
                axis=1, keepdims=True,
            )
            m = (col_i == jstar).astype(jnp.int32)
            acc = acc + gbits[:, t][:, None] * m
            cosm = jnp.where(m > 0, _NEG, cosm)
        for b in range(BATCH):
            a_scr[b] = ((acc >> b) & 1).astype(jnp.float32) * _NORM

    xb = x_ref[0]
    w = w_ref[...]
    h = jax.lax.dot_general(
        w, xb, (((0,), (0,)), ((), ())), preferred_element_type=jnp.float32
    )
    bias = b_ref[...]
    for bn in range(BATCH):
        hb = h[:, bn * NUM_NODES : (bn + 1) * NUM_NODES]
        ob = jax.lax.dot_general(
            hb, a_scr[bn], (((1,), (1,)), ((), ())),
            preferred_element_type=jnp.float32,
        )
        o_ref[0, :, bn * NUM_NODES : (bn + 1) * NUM_NODES] = ob + bias


def kernel(x, emb_table, weight, bias, logits):
    n_total = BATCH * NUM_NODES
    dl = (logits[:, 0] - logits[:, 1]).reshape(BATCH, NUM_NODES, TOPK)

    zero3 = lambda i: (0, 0, 0)
    out = pl.pallas_call(
        _fused_kernel,
        grid=(BATCH,),
        in_specs=[
            pl.BlockSpec((NUM_NODES, 64), lambda i: (0, 0)),
            pl.BlockSpec((BATCH, NUM_NODES, TOPK), zero3),
            pl.BlockSpec((BATCH, NUM_NODES, TOPK), zero3),
            pl.BlockSpec((1, SEQ_LEN, n_total), lambda i: (i, 0, 0)),
            pl.BlockSpec((SEQ_LEN, SEQ_LEN), lambda i: (0, 0)),
            pl.BlockSpec((SEQ_LEN, 1), lambda i: (0, 0)),
        ],
        out_specs=pl.BlockSpec((1, SEQ_LEN, n_total), lambda i: (i, 0, 0)),
        out_shape=jax.ShapeDtypeStruct((BATCH, SEQ_LEN, n_total), jnp.float32),
        scratch_shapes=[pltpu.VMEM((BATCH, NUM_NODES, NUM_NODES), jnp.float32)],
    )(emb_table, dl, jnp.asarray(_GD), x, weight, bias.reshape(SEQ_LEN, 1))
    return out
